# baseline (speedup 1.0000x reference)
import functools

import jax
import jax.numpy as jnp
from jax import lax
from jax.experimental import pallas as pl
from jax.experimental.pallas import tpu as pltpu
from jax.experimental.pallas import tpu_sc as plsc

_DIM = 300
_DIMP = 384
_TAIL = _DIM - 256
_NC = 2
_NS = 16
_NW = _NC * _NS
_CH = 128


def _make_gather(n_idx):
    assert n_idx % (_NW * _CH) == 0
    n_chunk = n_idx // (_NW * _CH)
    b_per_w = n_chunk * _CH
    mesh = plsc.VectorSubcoreMesh(core_axis_name="c", subcore_axis_name="s")

    @functools.partial(
        pl.kernel,
        mesh=mesh,
        out_type=jax.ShapeDtypeStruct((n_idx, _DIM), jnp.float32),
        scratch_types=[
            pltpu.VMEM((n_chunk, _CH), jnp.int32),
            pltpu.VMEM((_CH, _DIMP), jnp.float32),
            pltpu.VMEM((_CH, _TAIL), jnp.float32),
            pltpu.SemaphoreType.DMA,
        ],
    )
    def gather_kernel(idx_hbm, table_hbm, out_hbm, idx_v, buf, tails, gsem):
        wid = lax.axis_index("s") * _NC + lax.axis_index("c")
        base = pl.multiple_of(wid * b_per_w, 8)
        pltpu.sync_copy(idx_hbm.at[wid], idx_v)

        def body(j, carry):
            pltpu.async_copy(table_hbm.at[idx_v.at[j]], buf, gsem).wait()

            def crow(r, c):
                tails[r, pl.ds(0, 16)] = buf[r, pl.ds(256, 16)]
                tails[r, pl.ds(16, 16)] = buf[r, pl.ds(272, 16)]
                tails[r, pl.ds(_TAIL - 16, 16)] = buf[r, pl.ds(240 + _TAIL, 16)]
                return c

            lax.fori_loop(0, _CH, crow, 0)
            row0 = pl.multiple_of(base + j * _CH, 8)
            rows = out_hbm.at[pl.ds(row0, _CH)]
            pltpu.sync_copy(buf.at[:, pl.ds(0, 128)], rows.at[:, pl.ds(0, 128)])
            pltpu.sync_copy(buf.at[:, pl.ds(128, 128)], rows.at[:, pl.ds(128, 128)])
            pltpu.sync_copy(tails, rows.at[:, pl.ds(256, _TAIL)])
            return carry

        lax.fori_loop(0, n_chunk, body, 0)

    return gather_kernel


def kernel(idxes, table):
    batch, seq = idxes.shape
    n_idx = batch * seq
    idx3d = idxes.reshape(_NW, n_idx // (_NW * _CH), _CH).astype(jnp.int32)
    table_p = jnp.pad(table, ((0, 0), (0, _DIMP - _DIM)))
    out = _make_gather(n_idx)(idx3d, table_p)
    return out.reshape(batch, seq, _DIM)


# speedup vs baseline: 1.0974x; 1.0974x over previous
import functools

import jax
import jax.numpy as jnp
from jax import lax
from jax.experimental import pallas as pl
from jax.experimental.pallas import tpu as pltpu
from jax.experimental.pallas import tpu_sc as plsc

_DIM = 300
_DIMP = 384
_TAIL = _DIM - 256
_NC = 2
_NS = 16
_NW = _NC * _NS
_CH = 64


def _make_gather(n_idx):
    assert n_idx % (_NW * _CH * 2) == 0
    n_chunk = n_idx // (_NW * _CH)
    b_per_w = n_chunk * _CH
    mesh = plsc.VectorSubcoreMesh(core_axis_name="c", subcore_axis_name="s")

    @functools.partial(
        pl.kernel,
        mesh=mesh,
        out_type=jax.ShapeDtypeStruct((n_idx, _DIM), jnp.float32),
        scratch_types=[
            pltpu.VMEM((n_chunk, _CH), jnp.int32),
            pltpu.VMEM((2, _CH, _DIMP), jnp.float32),
            pltpu.VMEM((2, _CH, _TAIL), jnp.float32),
            pltpu.SemaphoreType.DMA,
            pltpu.SemaphoreType.DMA,
            pltpu.SemaphoreType.DMA,
        ],
    )
    def gather_kernel(idx_hbm, table_hbm, out_hbm, idx_v, buf, tails, gsem,
                      wsem0, wsem1):
        wid = lax.axis_index("s") * _NC + lax.axis_index("c")
        base = pl.multiple_of(wid * b_per_w, 8)
        pltpu.sync_copy(idx_hbm.at[wid], idx_v)
        wsems = (wsem0, wsem1)

        def start_gather(j, s):
            pltpu.async_copy(table_hbm.at[idx_v.at[j]], buf.at[s], gsem)

        def wait_gather(s):
            pltpu.make_async_copy(table_hbm.at[idx_v.at[0]], buf.at[s],
                                  gsem).wait()

        def out_rows(j):
            row0 = pl.multiple_of(base + j * _CH, 8)
            return out_hbm.at[pl.ds(row0, _CH)]

        def compact(s):
            def crow(r, c):
                tails[s, r, pl.ds(0, 16)] = buf[s, r, pl.ds(256, 16)]
                tails[s, r, pl.ds(16, 16)] = buf[s, r, pl.ds(272, 16)]
                tails[s, r, pl.ds(_TAIL - 16, 16)] = buf[s, r,
                                                         pl.ds(240 + _TAIL, 16)]
                return c
            lax.fori_loop(0, _CH, crow, 0, unroll=4)

        def start_writes(j, s):
            rows = out_rows(j)
            sem = wsems[s]
            pltpu.async_copy(buf.at[s, :, pl.ds(0, 128)],
                             rows.at[:, pl.ds(0, 128)], sem)
            pltpu.async_copy(buf.at[s, :, pl.ds(128, 128)],
                             rows.at[:, pl.ds(128, 128)], sem)
            pltpu.async_copy(tails.at[s], rows.at[:, pl.ds(256, _TAIL)], sem)

        def wait_writes(s):
            rows = out_rows(0)
            sem = wsems[s]
            pltpu.make_async_copy(buf.at[s, :, pl.ds(0, 128)],
                                  rows.at[:, pl.ds(0, 128)], sem).wait()
            pltpu.make_async_copy(buf.at[s, :, pl.ds(128, 128)],
                                  rows.at[:, pl.ds(128, 128)], sem).wait()
            pltpu.make_async_copy(tails.at[s],
                                  rows.at[:, pl.ds(256, _TAIL)], sem).wait()

        start_gather(0, 0)

        def body(t, carry):
            j0 = 2 * t
            j1 = j0 + 1
            wait_gather(0)

            @pl.when(t > 0)
            def _():
                wait_writes(1)

            start_gather(j1, 1)
            compact(0)
            start_writes(j0, 0)
            wait_gather(1)
            wait_writes(0)

            @pl.when(t < n_chunk // 2 - 1)
            def _():
                start_gather(j0 + 2, 0)

            compact(1)
            start_writes(j1, 1)
            return carry

        lax.fori_loop(0, n_chunk // 2, body, 0)
        wait_writes(1)

    return gather_kernel


def kernel(idxes, table):
    batch, seq = idxes.shape
    n_idx = batch * seq
    idx3d = idxes.reshape(_NW, n_idx // (_NW * _CH), _CH).astype(jnp.int32)
    table_p = jnp.pad(table, ((0, 0), (0, _DIMP - _DIM)))
    out = _make_gather(n_idx)(idx3d, table_p)
    return out.reshape(batch, seq, _DIM)


# speedup vs baseline: 1.8473x; 1.6834x over previous
import functools

import jax
import jax.numpy as jnp
from jax import lax
from jax.experimental import pallas as pl
from jax.experimental.pallas import tpu as pltpu
from jax.experimental.pallas import tpu_sc as plsc

_DIM = 300
_DIMP = 384
_TAIL = _DIM - 256
_NC = 2
_NS = 16
_NW = _NC * _NS
_CH = 64


def _make_gather(n_idx):
    assert n_idx % (_NW * _CH * 2) == 0
    n_chunk = n_idx // (_NW * _CH)
    b_per_w = n_chunk * _CH
    mesh = plsc.VectorSubcoreMesh(core_axis_name="c", subcore_axis_name="s")

    @functools.partial(
        pl.kernel,
        mesh=mesh,
        out_type=jax.ShapeDtypeStruct((n_idx, _DIM), jnp.float32),
        scratch_types=[
            pltpu.VMEM((n_chunk, _CH), jnp.int32),
            pltpu.VMEM((2, _CH, _DIMP), jnp.float32),
            pltpu.VMEM((2, _CH, _TAIL), jnp.float32),
            pltpu.SemaphoreType.DMA,
            pltpu.SemaphoreType.DMA,
            pltpu.SemaphoreType.DMA,
        ],
    )
    def gather_kernel(idx_hbm, table_hbm, out_hbm, idx_v, buf, tails, gsem,
                      wsem0, wsem1):
        wid = lax.axis_index("s") * _NC + lax.axis_index("c")
        base = pl.multiple_of(wid * b_per_w, 8)
        pltpu.sync_copy(idx_hbm.at[wid], idx_v)
        wsems = (wsem0, wsem1)

        def start_gather(j, s):
            pltpu.async_copy(table_hbm.at[idx_v.at[j]], buf.at[s], gsem)

        def wait_gather(s):
            pltpu.make_async_copy(table_hbm.at[idx_v.at[0]], buf.at[s],
                                  gsem).wait()

        def out_rows(j):
            row0 = pl.multiple_of(base + j * _CH, 8)
            return out_hbm.at[pl.ds(row0, _CH)]

        def compact(s):
            def crow(r, c):
                tails[s, r, pl.ds(0, 16)] = buf[s, r, pl.ds(256, 16)]
                tails[s, r, pl.ds(16, 16)] = buf[s, r, pl.ds(272, 16)]
                tails[s, r, pl.ds(_TAIL - 16, 16)] = buf[s, r,
                                                         pl.ds(240 + _TAIL, 16)]
                return c
            lax.fori_loop(0, _CH, crow, 0, unroll=4)

        def start_writes(j, s):
            rows = out_rows(j)
            sem = wsems[s]
            pltpu.async_copy(buf.at[s, :, pl.ds(0, 128)],
                             rows.at[:, pl.ds(0, 128)], sem)
            pltpu.async_copy(buf.at[s, :, pl.ds(128, 128)],
                             rows.at[:, pl.ds(128, 128)], sem)
            pltpu.async_copy(tails.at[s], rows.at[:, pl.ds(256, _TAIL)], sem)

        def wait_writes(s):
            rows = out_rows(0)
            sem = wsems[s]
            pltpu.make_async_copy(buf.at[s, :, pl.ds(0, 128)],
                                  rows.at[:, pl.ds(0, 128)], sem).wait()
            pltpu.make_async_copy(buf.at[s, :, pl.ds(128, 128)],
                                  rows.at[:, pl.ds(128, 128)], sem).wait()
            pltpu.make_async_copy(tails.at[s],
                                  rows.at[:, pl.ds(256, _TAIL)], sem).wait()

        start_gather(0, 0)

        def body(t, carry):
            j0 = 2 * t
            j1 = j0 + 1
            wait_gather(0)

            @pl.when(t > 0)
            def _():
                wait_writes(1)

            start_gather(j1, 1)
            compact(0)
            start_writes(j0, 0)
            wait_gather(1)
            wait_writes(0)

            @pl.when(t < n_chunk // 2 - 1)
            def _():
                start_gather(j0 + 2, 0)

            compact(1)
            start_writes(j1, 1)
            return carry

        lax.fori_loop(0, n_chunk // 2, body, 0)
        wait_writes(1)

    return gather_kernel


def kernel(idxes, table):
    batch, seq = idxes.shape
    n_idx = batch * seq
    idx3d = idxes.reshape(_NW, n_idx // (_NW * _CH), _CH).astype(jnp.int32)
    eye_pad = (jnp.arange(_DIM)[:, None] == jnp.arange(_DIMP)[None, :]
               ).astype(table.dtype)
    table_p = jax.lax.dot(table, eye_pad,
                          precision=jax.lax.Precision.HIGHEST)
    out = _make_gather(n_idx)(idx3d, table_p)
    return out.reshape(batch, seq, _DIM)


# speedup vs baseline: 1.8715x; 1.0131x over previous
import functools

import jax
import jax.numpy as jnp
from jax import lax
from jax.experimental import pallas as pl
from jax.experimental.pallas import tpu as pltpu
from jax.experimental.pallas import tpu_sc as plsc

_DIM = 300
_DIMP = 384
_TAIL = _DIM - 256
_NC = 2
_NS = 16
_NW = _NC * _NS
_CH = 80


def _make_gather(n_idx):
    assert n_idx % (_NW * _CH * 2) == 0
    n_chunk = n_idx // (_NW * _CH)
    b_per_w = n_chunk * _CH
    mesh = plsc.VectorSubcoreMesh(core_axis_name="c", subcore_axis_name="s")

    @functools.partial(
        pl.kernel,
        mesh=mesh,
        out_type=jax.ShapeDtypeStruct((n_idx, _DIM), jnp.float32),
        scratch_types=[
            pltpu.VMEM((n_chunk, _CH), jnp.int32),
            pltpu.VMEM((2, _CH, _DIMP), jnp.float32),
            pltpu.VMEM((2, _CH, _TAIL), jnp.float32),
            pltpu.SemaphoreType.DMA,
            pltpu.SemaphoreType.DMA,
            pltpu.SemaphoreType.DMA,
        ],
    )
    def gather_kernel(idx_hbm, table_hbm, out_hbm, idx_v, buf, tails, gsem,
                      wsem0, wsem1):
        wid = lax.axis_index("s") * _NC + lax.axis_index("c")
        base = pl.multiple_of(wid * b_per_w, 8)
        pltpu.sync_copy(idx_hbm.at[wid], idx_v)
        wsems = (wsem0, wsem1)

        def start_gather(j, s):
            pltpu.async_copy(table_hbm.at[idx_v.at[j]], buf.at[s], gsem)

        def wait_gather(s):
            pltpu.make_async_copy(table_hbm.at[idx_v.at[0]], buf.at[s],
                                  gsem).wait()

        def out_rows(j):
            row0 = pl.multiple_of(base + j * _CH, 8)
            return out_hbm.at[pl.ds(row0, _CH)]

        def compact(s):
            def crow(r, c):
                tails[s, r, pl.ds(0, 16)] = buf[s, r, pl.ds(256, 16)]
                tails[s, r, pl.ds(16, 16)] = buf[s, r, pl.ds(272, 16)]
                tails[s, r, pl.ds(_TAIL - 16, 16)] = buf[s, r,
                                                         pl.ds(240 + _TAIL, 16)]
                return c
            lax.fori_loop(0, _CH, crow, 0, unroll=4)

        def start_writes(j, s):
            rows = out_rows(j)
            sem = wsems[s]
            pltpu.async_copy(buf.at[s, :, pl.ds(0, 128)],
                             rows.at[:, pl.ds(0, 128)], sem)
            pltpu.async_copy(buf.at[s, :, pl.ds(128, 128)],
                             rows.at[:, pl.ds(128, 128)], sem)
            pltpu.async_copy(tails.at[s], rows.at[:, pl.ds(256, _TAIL)], sem)

        def wait_writes(s):
            rows = out_rows(0)
            sem = wsems[s]
            pltpu.make_async_copy(buf.at[s, :, pl.ds(0, 128)],
                                  rows.at[:, pl.ds(0, 128)], sem).wait()
            pltpu.make_async_copy(buf.at[s, :, pl.ds(128, 128)],
                                  rows.at[:, pl.ds(128, 128)], sem).wait()
            pltpu.make_async_copy(tails.at[s],
                                  rows.at[:, pl.ds(256, _TAIL)], sem).wait()

        start_gather(0, 0)

        def body(t, carry):
            j0 = 2 * t
            j1 = j0 + 1
            wait_gather(0)

            @pl.when(t > 0)
            def _():
                wait_writes(1)

            start_gather(j1, 1)
            compact(0)
            start_writes(j0, 0)
            wait_gather(1)
            wait_writes(0)

            @pl.when(t < n_chunk // 2 - 1)
            def _():
                start_gather(j0 + 2, 0)

            compact(1)
            start_writes(j1, 1)
            return carry

        lax.fori_loop(0, n_chunk // 2, body, 0)
        wait_writes(1)

    return gather_kernel


def kernel(idxes, table):
    batch, seq = idxes.shape
    n_idx = batch * seq
    idx3d = idxes.reshape(_NW, n_idx // (_NW * _CH), _CH).astype(jnp.int32)
    eye_pad = (jnp.arange(_DIM)[:, None] == jnp.arange(_DIMP)[None, :]
               ).astype(table.dtype)
    table_p = jax.lax.dot(table, eye_pad,
                          precision=jax.lax.Precision.HIGHEST)
    out = _make_gather(n_idx)(idx3d, table_p)
    return out.reshape(batch, seq, _DIM)
